# baseline (speedup 1.0000x reference)
import functools

import jax
import jax.numpy as jnp
from jax.experimental import pallas as pl
from jax.experimental.pallas import tpu as pltpu

N_EXPERTS = 8
TOP_K = 2
DIM = 2048
E = N_EXPERTS - 1


def _fused_moe_kernel(x_ref, wg_ref, ws_ref, bias_ref, out_ref):
    x = x_ref[...]
    xb = x.astype(jnp.bfloat16)
    logits = jnp.dot(xb, wg_ref[...], preferred_element_type=jnp.float32)
    m = jnp.max(logits, axis=-1, keepdims=True)
    e = jnp.exp(logits - m)
    scores = e / jnp.sum(e, axis=-1, keepdims=True)
    m1 = jnp.max(scores, axis=-1, keepdims=True)
    col = jax.lax.broadcasted_iota(jnp.int32, scores.shape, 1)
    first = jnp.min(jnp.where(scores == m1, col, E), axis=-1, keepdims=True)
    masked = jnp.where(col == first, -jnp.inf, scores)
    m2 = jnp.max(masked, axis=-1, keepdims=True)
    s = m1 + m2
    wsum = m1 / s + m2 / s
    acc = jnp.dot(xb, ws_ref[...], preferred_element_type=jnp.float32)
    out_ref[...] = acc + x * wsum + bias_ref[...]


@functools.partial(jax.jit, static_argnames=())
def kernel(x, cond, mask, Wg, Ws, bias):
    b, n, d = x.shape
    T = b * n
    x_flat = x.reshape(T, d)
    TM = 512
    grid = (T // TM,)
    out = pl.pallas_call(
        _fused_moe_kernel,
        grid=grid,
        in_specs=[
            pl.BlockSpec((TM, d), lambda i: (i, 0)),
            pl.BlockSpec((d, E), lambda i: (0, 0)),
            pl.BlockSpec((d, d), lambda i: (0, 0)),
            pl.BlockSpec((1, d), lambda i: (0, 0)),
        ],
        out_specs=pl.BlockSpec((TM, d), lambda i: (i, 0)),
        out_shape=jax.ShapeDtypeStruct((T, d), jnp.float32),
    )(x_flat, Wg.astype(jnp.bfloat16), Ws.astype(jnp.bfloat16),
      bias.reshape(1, d))
    return out.reshape(b, n, d)


# speedup vs baseline: 1.2218x; 1.2218x over previous
import jax
import jax.numpy as jnp
from jax.experimental import pallas as pl


def _fused_moe_kernel(x_ref, ws_ref, bias_ref, out_ref):
    x = x_ref[...]
    acc = jnp.dot(x, ws_ref[...], preferred_element_type=jnp.float32)
    out_ref[...] = acc + x + bias_ref[...]


def kernel(x, cond, mask, Wg, Ws, bias):
    b, n, d = x.shape
    T = b * n
    x_flat = x.reshape(T, d)
    TM = 512
    grid = (T // TM,)
    out = pl.pallas_call(
        _fused_moe_kernel,
        grid=grid,
        in_specs=[
            pl.BlockSpec((TM, d), lambda i: (i, 0)),
            pl.BlockSpec((d, d), lambda i: (0, 0)),
            pl.BlockSpec((1, d), lambda i: (0, 0)),
        ],
        out_specs=pl.BlockSpec((TM, d), lambda i: (i, 0)),
        out_shape=jax.ShapeDtypeStruct((T, d), jnp.float32),
    )(x_flat, Ws, bias.reshape(1, d))
    return out.reshape(b, n, d)


# speedup vs baseline: 1.2225x; 1.0006x over previous
import jax
import jax.numpy as jnp
from jax.experimental import pallas as pl
from jax.experimental.pallas import tpu as pltpu


def _fused_moe_kernel(x_ref, ws_ref, bias_ref, out_ref):
    x = x_ref[...]
    acc = jnp.dot(x, ws_ref[...], preferred_element_type=jnp.float32)
    out_ref[...] = acc + x + bias_ref[...]


def kernel(x, cond, mask, Wg, Ws, bias):
    b, n, d = x.shape
    T = b * n
    x_flat = x.reshape(T, d)
    TM = 512
    grid = (T // TM,)
    out = pl.pallas_call(
        _fused_moe_kernel,
        grid=grid,
        in_specs=[
            pl.BlockSpec((TM, d), lambda i: (i, 0)),
            pl.BlockSpec((d, d), lambda i: (0, 0)),
            pl.BlockSpec((1, d), lambda i: (0, 0)),
        ],
        out_specs=pl.BlockSpec((TM, d), lambda i: (i, 0)),
        out_shape=jax.ShapeDtypeStruct((T, d), jnp.float32),
        compiler_params=pltpu.CompilerParams(
            dimension_semantics=("parallel",)),
    )(x_flat, Ws, bias.reshape(1, d))
    return out.reshape(b, n, d)
